# baseline (speedup 1.0000x reference)
import functools

import jax
import jax.numpy as jnp
from jax import lax
from jax.experimental import pallas as pl
from jax.experimental.pallas import tpu as pltpu
from jax.experimental.pallas import tpu_sc as plsc

RES = 128
TS = 524288
D = 8
C = 1000
NW = 32
NT = 16
CI = (C + 15) // 16
GROUPS = [(j * 128, 128) for j in range(C // 128)] + [(C - C % 128, C % 128)]
TROWS = TS // NT
TCHUNK = 1024


@functools.lru_cache(maxsize=None)
def _build(n):
    assert n % C == 0
    nchunks = n // C
    mesh = plsc.VectorSubcoreMesh(core_axis_name="c", subcore_axis_name="s")

    @functools.partial(
        pl.kernel,
        mesh=mesh,
        compiler_params=pltpu.CompilerParams(needs_layout_passes=False,
                                             use_tc_tiling_on_sc=False),
        out_type=jax.ShapeDtypeStruct((n * D,), jnp.float32),
        scratch_types=[
            pltpu.HBM((2, TS, D), jnp.float32),
            pltpu.VMEM((3 * C + 16,), jnp.float32),
            pltpu.VMEM((len(GROUPS), 128), jnp.int32),
            pltpu.VMEM((CI * 16,), jnp.int32),
            pltpu.VMEM((C, D), jnp.float32),
            pltpu.VMEM(((CI * 16) * D,), jnp.float32),
            pltpu.VMEM((TCHUNK * D,), jnp.float32),
            pltpu.VMEM((TCHUNK, D), jnp.float32),
            pltpu.VMEM((16,), jnp.float32),
            pltpu.SemaphoreType.DMA,
        ],
    )
    def lookup(pos_hbm, tabf_hbm, out_hbm, tabs_hbm, pos_v, idx_v,
               msk_v, rows_v, stage_v, tf_v, t2_v, hot_v, sem):
        tab0_hbm = tabs_hbm.at[0]
        tab1_hbm = tabs_hbm.at[1]
        cid = lax.axis_index("c")
        sid = lax.axis_index("s")
        wid = sid * 2 + cid
        lane = lax.iota(jnp.int32, 16)
        col3 = lane * 3
        rsub = lane >> 3
        csub = lane & 7

        def stage_table(tab_hbm):
            def chunk(k, carry):
                r0 = sid * TROWS + k * TCHUNK
                pltpu.sync_copy(tabf_hbm.at[pl.ds(r0 * D, TCHUNK * D)],
                                tf_v)

                def body(t, carry2):
                    v = tf_v[pl.ds(t * 16, 16)]
                    plsc.store_scatter(t2_v, [t * 2 + rsub, csub], v)
                    return carry2

                lax.fori_loop(0, TCHUNK * D // 16, body, None)
                pltpu.sync_copy(t2_v, tab_hbm.at[pl.ds(r0, TCHUNK)])
                return carry

            lax.fori_loop(0, TROWS // TCHUNK, chunk, None)

        @pl.when(cid == 0)
        def _():
            stage_table(tab0_hbm)

        @pl.when(cid == 1)
        def _():
            stage_table(tab1_hbm)

        plsc.subcore_barrier()

        pos_v[pl.ds(3 * C, 16)] = jnp.zeros((16,), jnp.float32)
        pltpu.sync_copy(tabf_hbm.at[pl.ds((TS - 2) * D, 16)], hot_v)
        colf = [jnp.full((16,), f, jnp.int32) for f in range(D)]
        hotf = [plsc.load_gather(hot_v, [jnp.full((16,), D + f, jnp.int32)])
                for f in range(D)]

        def run_chunks(table_hbm):
            def chunk_body(t, carry):
                g = t * NW + wid

                @pl.when(g < nchunks)
                def _():
                    base = g * C
                    pltpu.sync_copy(pos_hbm.at[pl.ds(base * 3, 3 * C)],
                                    pos_v.at[pl.ds(0, 3 * C)])

                    def compute(i, carry2):
                        off = i * 48

                        def comp(c):
                            p = plsc.load_gather(pos_v, [col3 + (off + c)])
                            ps = (p + 1.0) * 64.0
                            return jnp.minimum(jnp.maximum(ps, 0.0),
                                               float(RES - 1))

                        f = (comp(0) * float(RES * RES)
                             + comp(1) * float(RES) + comp(2))
                        ii = f.astype(jnp.int32)
                        ii = jnp.minimum(jnp.maximum(ii, 0), TS - 1)
                        m = ii < TS - 1
                        pv = lane + i * 16
                        dmy = (base + pv) & (TS - 1)
                        idx_v[i // 8, pl.ds((i % 8) * 16, 16)] = jnp.where(
                            m, ii, dmy)
                        msk_v[pl.ds(i * 16, 16)] = m.astype(jnp.int32)
                        return carry2

                    lax.fori_loop(0, CI, compute, None)

                    copies = [
                        pltpu.async_copy(
                            table_hbm.at[idx_v.at[j, pl.ds(0, cnt)]],
                            rows_v.at[pl.ds(dst0, cnt)],
                            sem)
                        for j, (dst0, cnt) in enumerate(GROUPS)
                    ]
                    for cp in copies:
                        cp.wait()

                    def merge(i, carry2):
                        mb = msk_v[pl.ds(i * 16, 16)] > 0
                        pv = lane + i * 16
                        inb = pv < C
                        pv8 = pv * D
                        for f in range(D):
                            gf = plsc.load_gather(rows_v, [pv, colf[f]],
                                                  mask=inb)
                            v = jnp.where(mb, gf, hotf[f])
                            plsc.store_scatter(stage_v, [pv8 + f], v,
                                               mask=inb)
                        return carry2

                    lax.fori_loop(0, CI, merge, None)
                    pltpu.sync_copy(stage_v.at[pl.ds(0, C * D)],
                                    out_hbm.at[pl.ds(base * D, C * D)])

                return carry

            lax.fori_loop(0, (nchunks + NW - 1) // NW, chunk_body, None)

        @pl.when(cid == 0)
        def _():
            run_chunks(tab0_hbm)

        @pl.when(cid == 1)
        def _():
            run_chunks(tab1_hbm)

    return lookup


def kernel(positions, table):
    n = positions.shape[0]
    out_flat = _build(n)(positions.reshape(-1), table.reshape(-1))
    return out_flat.reshape(n, D)


# speedup vs baseline: 1.0001x; 1.0001x over previous
import functools

import jax
import jax.numpy as jnp
from jax import lax
from jax.experimental import pallas as pl
from jax.experimental.pallas import tpu as pltpu
from jax.experimental.pallas import tpu_sc as plsc

RES = 128
TS = 524288
D = 8
C = 1000
NW = 32
NT = 16
CI = (C + 15) // 16
GROUPS = [(j * 128, 128) for j in range(C // 128)] + [(C - C % 128, C % 128)]
TROWS = TS // NT
TCHUNK = 1024


@functools.lru_cache(maxsize=None)
def _build(n):
    assert n % C == 0
    nchunks = n // C
    mesh = plsc.VectorSubcoreMesh(core_axis_name="c", subcore_axis_name="s")

    @functools.partial(
        pl.kernel,
        mesh=mesh,
        compiler_params=pltpu.CompilerParams(needs_layout_passes=False,
                                             use_tc_tiling_on_sc=False),
        out_type=jax.ShapeDtypeStruct((n * D,), jnp.float32),
        scratch_types=[
            pltpu.HBM((2, TS, D), jnp.float32),
            pltpu.VMEM((3 * C + 16,), jnp.float32),
            pltpu.VMEM((len(GROUPS), 128), jnp.int32),
            pltpu.VMEM((CI * 16,), jnp.int32),
            pltpu.VMEM((C, D), jnp.float32),
            pltpu.VMEM(((CI * 16) * D,), jnp.float32),
            pltpu.VMEM((TCHUNK * D,), jnp.float32),
            pltpu.VMEM((TCHUNK, D), jnp.float32),
            pltpu.VMEM((16,), jnp.float32),
            pltpu.SemaphoreType.DMA,
        ],
    )
    def lookup(pos_hbm, tabf_hbm, out_hbm, tabs_hbm, pos_v, idx_v,
               msk_v, rows_v, stage_v, tf_v, t2_v, hot_v, sem):
        tab0_hbm = tabs_hbm.at[0]
        tab1_hbm = tabs_hbm.at[1]
        cid = lax.axis_index("c")
        sid = lax.axis_index("s")
        wid = sid * 2 + cid
        lane = lax.iota(jnp.int32, 16)
        col3 = lane * 3
        rsub = lane >> 3
        csub = lane & 7

        def stage_table(tab_hbm):
            def chunk(k, carry):
                r0 = sid * TROWS + k * TCHUNK
                pltpu.sync_copy(tabf_hbm.at[pl.ds(r0 * D, TCHUNK * D)],
                                tf_v)

                def body(t, carry2):
                    v = tf_v[pl.ds(t * 16, 16)]
                    plsc.store_scatter(t2_v, [t * 2 + rsub, csub], v)
                    return carry2

                lax.fori_loop(0, TCHUNK * D // 16, body, None)
                pltpu.sync_copy(t2_v, tab_hbm.at[pl.ds(r0, TCHUNK)])
                return carry

            lax.fori_loop(0, TROWS // TCHUNK, chunk, None)

        @pl.when(cid == 0)
        def _():
            stage_table(tab0_hbm)

        @pl.when(cid == 1)
        def _():
            stage_table(tab1_hbm)

        plsc.subcore_barrier()

        pos_v[pl.ds(3 * C, 16)] = jnp.zeros((16,), jnp.float32)
        pltpu.sync_copy(tabf_hbm.at[pl.ds((TS - 2) * D, 16)], hot_v)
        colf = [jnp.full((16,), f, jnp.int32) for f in range(D)]
        hotf = [plsc.load_gather(hot_v, [jnp.full((16,), D + f, jnp.int32)])
                for f in range(D)]

        def run_chunks(table_hbm):
            def chunk_body(t, carry):
                g = t * NW + wid

                @pl.when(g < nchunks)
                def _():
                    base = g * C
                    pltpu.sync_copy(pos_hbm.at[pl.ds(base * 3, 3 * C)],
                                    pos_v.at[pl.ds(0, 3 * C)])

                    def compute(i, carry2):
                        off = i * 48

                        def comp(c):
                            p = plsc.load_gather(pos_v, [col3 + (off + c)])
                            ps = (p + 1.0) * 64.0
                            return jnp.minimum(jnp.maximum(ps, 0.0),
                                               float(RES - 1))

                        f = (comp(0) * float(RES * RES)
                             + comp(1) * float(RES) + comp(2))
                        ii = f.astype(jnp.int32)
                        ii = jnp.minimum(jnp.maximum(ii, 0), TS - 1)
                        m = ii < TS - 1
                        pv = lane + i * 16
                        dmy = (base + pv) & (TS - 1)
                        idx_v[i // 8, pl.ds((i % 8) * 16, 16)] = jnp.where(
                            m, ii, dmy)
                        msk_v[pl.ds(i * 16, 16)] = m.astype(jnp.int32)
                        return carry2

                    lax.fori_loop(0, CI, compute, None)

                    copies = [
                        pltpu.async_copy(
                            table_hbm.at[idx_v.at[j, pl.ds(0, cnt)]],
                            rows_v.at[pl.ds(dst0, cnt)],
                            sem)
                        for j, (dst0, cnt) in enumerate(GROUPS)
                    ]
                    for cp in copies:
                        cp.wait()

                    def merge(i, carry2):
                        mb = msk_v[pl.ds(i * 16, 16)] > 0
                        pv = lane + i * 16
                        inb = pv < C
                        pv8 = pv * D
                        for f in range(D):
                            gf = plsc.load_gather(rows_v, [pv, colf[f]],
                                                  mask=inb)
                            v = jnp.where(mb, gf, hotf[f])
                            plsc.store_scatter(stage_v, [pv8 + f], v,
                                               mask=inb)
                        return carry2

                    lax.fori_loop(0, CI, merge, None)
                    pltpu.sync_copy(stage_v.at[pl.ds(0, C * D)],
                                    out_hbm.at[pl.ds(base * D, C * D)])

                return carry

            lax.fori_loop(0, (nchunks + NW - 1) // NW, chunk_body, None)

        @pl.when(cid == 0)
        def _():
            run_chunks(tab0_hbm)

        @pl.when(cid == 1)
        def _():
            run_chunks(tab1_hbm)

    return lookup


def kernel(positions, table):
    n = positions.shape[0]
    pos_flat, tab_flat = lax.optimization_barrier(
        (positions.reshape(-1), table.reshape(-1)))
    out_flat = _build(n)(pos_flat, tab_flat)
    return out_flat.reshape(n, D)


# speedup vs baseline: 2.2482x; 2.2480x over previous
import functools

import jax
import jax.numpy as jnp
from jax import lax
from jax.experimental import pallas as pl
from jax.experimental.pallas import tpu as pltpu
from jax.experimental.pallas import tpu_sc as plsc

RES = 128
TS = 524288
D = 8
C = 1000
NW = 32
NT = 16
CI = (C + 15) // 16
GROUPS = [(j * 128, 128) for j in range(C // 128)] + [(C - C % 128, C % 128)]
TROWS = TS // NT
TCHUNK = 1024


@functools.lru_cache(maxsize=None)
def _build(n):
    assert n % C == 0
    nchunks = n // C
    mesh = plsc.VectorSubcoreMesh(core_axis_name="c", subcore_axis_name="s")

    @functools.partial(
        pl.kernel,
        mesh=mesh,
        compiler_params=pltpu.CompilerParams(needs_layout_passes=False,
                                             use_tc_tiling_on_sc=False),
        out_type=jax.ShapeDtypeStruct((n * D,), jnp.float32),
        scratch_types=[
            pltpu.HBM((2, TS, D), jnp.float32),
            pltpu.VMEM((C + 16,), jnp.int32),
            pltpu.VMEM((len(GROUPS), 128), jnp.int32),
            pltpu.VMEM((CI * 16,), jnp.int32),
            pltpu.VMEM((C, D), jnp.float32),
            pltpu.VMEM(((CI * 16) * D,), jnp.float32),
            pltpu.VMEM((TCHUNK * D,), jnp.float32),
            pltpu.VMEM((TCHUNK, D), jnp.float32),
            pltpu.VMEM((16,), jnp.float32),
            pltpu.SemaphoreType.DMA,
        ],
    )
    def lookup(idx_hbm, tabf_hbm, out_hbm, tabs_hbm, pin_v, idx_v,
               msk_v, rows_v, stage_v, tf_v, t2_v, hot_v, sem):
        tab0_hbm = tabs_hbm.at[0]
        tab1_hbm = tabs_hbm.at[1]
        cid = lax.axis_index("c")
        sid = lax.axis_index("s")
        wid = sid * 2 + cid
        lane = lax.iota(jnp.int32, 16)
        col3 = lane * 3
        rsub = lane >> 3
        csub = lane & 7

        def stage_table(tab_hbm):
            def chunk(k, carry):
                r0 = sid * TROWS + k * TCHUNK
                pltpu.sync_copy(tabf_hbm.at[pl.ds(r0 * D, TCHUNK * D)],
                                tf_v)

                def body(t, carry2):
                    v = tf_v[pl.ds(t * 16, 16)]
                    plsc.store_scatter(t2_v, [t * 2 + rsub, csub], v)
                    return carry2

                lax.fori_loop(0, TCHUNK * D // 16, body, None)
                pltpu.sync_copy(t2_v, tab_hbm.at[pl.ds(r0, TCHUNK)])
                return carry

            lax.fori_loop(0, TROWS // TCHUNK, chunk, None)

        @pl.when(cid == 0)
        def _():
            stage_table(tab0_hbm)

        @pl.when(cid == 1)
        def _():
            stage_table(tab1_hbm)

        plsc.subcore_barrier()

        pin_v[pl.ds(C, 16)] = jnp.zeros((16,), jnp.int32)
        pltpu.sync_copy(tabf_hbm.at[pl.ds((TS - 2) * D, 16)], hot_v)
        colf = [jnp.full((16,), f, jnp.int32) for f in range(D)]
        hotf = [plsc.load_gather(hot_v, [jnp.full((16,), D + f, jnp.int32)])
                for f in range(D)]

        def run_chunks(table_hbm):
            def chunk_body(t, carry):
                g = t * NW + wid

                @pl.when(g < nchunks)
                def _():
                    base = g * C
                    pltpu.sync_copy(idx_hbm.at[pl.ds(base, C)],
                                    pin_v.at[pl.ds(0, C)])

                    def compute(i, carry2):
                        ii = pin_v[pl.ds(i * 16, 16)]
                        m = ii < TS - 1
                        pv = lane + i * 16
                        dmy = (base + pv) & (TS - 1)
                        idx_v[i // 8, pl.ds((i % 8) * 16, 16)] = jnp.where(
                            m, ii, dmy)
                        msk_v[pl.ds(i * 16, 16)] = m.astype(jnp.int32)
                        return carry2

                    lax.fori_loop(0, CI, compute, None)

                    copies = [
                        pltpu.async_copy(
                            table_hbm.at[idx_v.at[j, pl.ds(0, cnt)]],
                            rows_v.at[pl.ds(dst0, cnt)],
                            sem)
                        for j, (dst0, cnt) in enumerate(GROUPS)
                    ]
                    for cp in copies:
                        cp.wait()

                    def merge(i, carry2):
                        mb = msk_v[pl.ds(i * 16, 16)] > 0
                        pv = lane + i * 16
                        inb = pv < C
                        pv8 = pv * D
                        for f in range(D):
                            gf = plsc.load_gather(rows_v, [pv, colf[f]],
                                                  mask=inb)
                            v = jnp.where(mb, gf, hotf[f])
                            plsc.store_scatter(stage_v, [pv8 + f], v,
                                               mask=inb)
                        return carry2

                    lax.fori_loop(0, CI, merge, None)
                    pltpu.sync_copy(stage_v.at[pl.ds(0, C * D)],
                                    out_hbm.at[pl.ds(base * D, C * D)])

                return carry

            lax.fori_loop(0, (nchunks + NW - 1) // NW, chunk_body, None)

        @pl.when(cid == 0)
        def _():
            run_chunks(tab0_hbm)

        @pl.when(cid == 1)
        def _():
            run_chunks(tab1_hbm)

    return lookup


@functools.lru_cache(maxsize=None)
def _build_idx(n, blk):
    def body(x_ref, o_ref):
        x = x_ref[...]
        ps = (x + 1.0) * 0.5 * float(RES)
        ps = jnp.minimum(jnp.maximum(ps, 0.0), float(RES - 1))
        f = (ps[:, 0] * float(RES * RES) + ps[:, 1] * float(RES)) + ps[:, 2]
        ii = f.astype(jnp.int32)
        o_ref[...] = jnp.minimum(jnp.maximum(ii, 0), TS - 1)

    return pl.pallas_call(
        body,
        out_shape=jax.ShapeDtypeStruct((n,), jnp.int32),
        grid=(pl.cdiv(n, blk),),
        in_specs=[pl.BlockSpec((blk, 3), lambda i: (i, 0))],
        out_specs=pl.BlockSpec((blk,), lambda i: (i,)),
    )


def kernel(positions, table):
    n = positions.shape[0]
    idx = _build_idx(n, 8192)(positions)
    out_flat = _build(n)(idx, table.reshape(-1))
    return out_flat.reshape(n, D)
